# Initial kernel scaffold; baseline (speedup 1.0000x reference)
import jax
import jax.numpy as jnp
from jax.experimental import pallas as pl


def kernel(feat, edge_index, W):
    raise NotImplementedError("write your pallas kernel here")


# speedup vs baseline: 6.0485x; 6.0485x over previous
import functools

import jax
import jax.numpy as jnp
from jax import lax
from jax.experimental import pallas as pl
from jax.experimental.pallas import tpu as pltpu
from jax.experimental.pallas import tpu_sc as plsc

N = 10000
E = 320000
D = 128
DH = D // 2

NC = 2
NS = 16
EPT = E // NS
CHUNK = 80
NCHUNK = EPT // CHUNK
NP = 10240
RPT = NP // NS
ZROWS = 128
DEGW = 16


def _sc_aggregate(src, dst, feat_cols):
    mesh = plsc.VectorSubcoreMesh(core_axis_name="c", subcore_axis_name="s")

    @functools.partial(
        pl.kernel,
        out_type=[
            jax.ShapeDtypeStruct((NC, NP, DH), jnp.float32),
            jax.ShapeDtypeStruct((NP, DEGW), jnp.float32),
        ],
        mesh=mesh,
        scratch_types=[
            pltpu.VMEM((NCHUNK, CHUNK), jnp.int32),
            pltpu.VMEM((NCHUNK, CHUNK), jnp.int32),
            pltpu.VMEM((CHUNK, DH), jnp.float32),
            pltpu.VMEM((CHUNK, DEGW), jnp.float32),
            pltpu.VMEM((ZROWS, DH), jnp.float32),
            pltpu.VMEM((RPT, DEGW), jnp.float32),
            pltpu.VMEM_SHARED((NP, DH), jnp.float32),
            pltpu.VMEM_SHARED((NP, DEGW), jnp.float32),
            pltpu.SemaphoreType.DMA,
        ],
        compiler_params=pltpu.CompilerParams(use_tc_tiling_on_sc=False),
    )
    def k(src_hbm, dst_hbm, feat_hbm, sum_out, deg_out,
          src_v, dst_v, rows_v, ones_v, zsum_v, zdeg_v, acc_sh, deg_sh, sem):
        cid = lax.axis_index("c")
        sid = lax.axis_index("s")
        row0 = sid * RPT

        zero16 = jnp.zeros((16,), jnp.float32)
        one16 = jnp.ones((16,), jnp.float32)

        def fill_ones(i, carry):
            ones_v[i] = one16
            return carry
        lax.fori_loop(0, CHUNK, fill_ones, 0)

        def fill_zsum(i, carry):
            r = i // (DH // 16)
            c = lax.rem(i, DH // 16)
            zsum_v[r, pl.ds(c * 16, 16)] = zero16
            return carry
        lax.fori_loop(0, ZROWS * (DH // 16), fill_zsum, 0)

        def fill_zdeg(i, carry):
            zdeg_v[i] = zero16
            return carry
        lax.fori_loop(0, RPT, fill_zdeg, 0)

        def zacc(t, carry):
            pltpu.sync_copy(zsum_v, acc_sh.at[pl.ds(row0 + t * ZROWS, ZROWS)])
            return carry
        lax.fori_loop(0, RPT // ZROWS, zacc, 0)
        pltpu.sync_copy(zdeg_v, deg_sh.at[pl.ds(row0, RPT)])

        pltpu.sync_copy(src_hbm.at[sid], src_v)
        pltpu.sync_copy(dst_hbm.at[sid], dst_v)

        plsc.subcore_barrier()

        @pl.when(cid == 0)
        def _():
            def body(j, carry):
                pltpu.async_copy(feat_hbm.at[0].at[src_v.at[j]],
                                 rows_v, sem).wait()
                pltpu.sync_copy(rows_v, acc_sh.at[dst_v.at[j]], add=True)
                pltpu.sync_copy(ones_v, deg_sh.at[dst_v.at[j]], add=True)
                return carry
            lax.fori_loop(0, NCHUNK, body, 0)

        @pl.when(cid == 1)
        def _():
            def body(j, carry):
                pltpu.async_copy(feat_hbm.at[1].at[src_v.at[j]],
                                 rows_v, sem).wait()
                pltpu.sync_copy(rows_v, acc_sh.at[dst_v.at[j]], add=True)
                return carry
            lax.fori_loop(0, NCHUNK, body, 0)

        plsc.subcore_barrier()

        pltpu.sync_copy(acc_sh.at[pl.ds(row0, RPT)],
                        sum_out.at[cid, pl.ds(row0, RPT)])

        @pl.when(cid == 0)
        def _():
            pltpu.sync_copy(deg_sh.at[pl.ds(row0, RPT)],
                            deg_out.at[pl.ds(row0, RPT)])

    return k(src, dst, feat_cols)


def _tc_finalize(sums, degs, W):
    R = 1000

    def body(s_ref, d_ref, w_ref, o_ref):
        s = jnp.concatenate([s_ref[0], s_ref[1]], axis=1)
        deg = d_ref[:, :1]
        rst = s / jnp.maximum(deg, 1.0)
        out = jnp.dot(rst, w_ref[...], preferred_element_type=jnp.float32)
        o_ref[...] = jnp.maximum(out, 0.0)

    return pl.pallas_call(
        body,
        grid=(N // R,),
        in_specs=[
            pl.BlockSpec((NC, R, DH), lambda i: (0, i, 0)),
            pl.BlockSpec((R, DEGW), lambda i: (i, 0)),
            pl.BlockSpec((D, D), lambda i: (0, 0)),
        ],
        out_specs=pl.BlockSpec((R, D), lambda i: (i, 0)),
        out_shape=jax.ShapeDtypeStruct((N, D), jnp.float32),
    )(sums, degs, W)


@jax.jit
def kernel(feat, edge_index, W):
    src = edge_index[0].reshape(NS, NCHUNK, CHUNK)
    dst = edge_index[1].reshape(NS, NCHUNK, CHUNK)
    feat_cols = jnp.stack([feat[:, :DH], feat[:, DH:]])
    sums, degs = _sc_aggregate(src, dst, feat_cols)
    return _tc_finalize(sums, degs, W)


# speedup vs baseline: 7.8527x; 1.2983x over previous
import functools

import jax
import jax.numpy as jnp
from jax import lax
from jax.experimental import pallas as pl
from jax.experimental.pallas import tpu as pltpu
from jax.experimental.pallas import tpu_sc as plsc

N = 10000
E = 320000
D = 128
DH = D // 2

NC = 2
NS = 16
EPT = E // NS
CHUNK = 80
NCHUNK = EPT // CHUNK
NP = 10240
RPT = NP // NS
ZROWS = 128
DEGW = 16


def _sc_aggregate(src, dst, feat_cols):
    mesh = plsc.VectorSubcoreMesh(core_axis_name="c", subcore_axis_name="s")

    @functools.partial(
        pl.kernel,
        out_type=[
            jax.ShapeDtypeStruct((NC, NP, DH), jnp.float32),
            jax.ShapeDtypeStruct((NP, DEGW), jnp.float32),
        ],
        mesh=mesh,
        scratch_types=[
            pltpu.VMEM((NCHUNK, CHUNK), jnp.int32),
            pltpu.VMEM((NCHUNK, CHUNK), jnp.int32),
            pltpu.VMEM((CHUNK, DH), jnp.float32),
            pltpu.VMEM((CHUNK, DH), jnp.float32),
            pltpu.VMEM((CHUNK, DEGW), jnp.float32),
            pltpu.VMEM((ZROWS, DH), jnp.float32),
            pltpu.VMEM((RPT, DEGW), jnp.float32),
            pltpu.VMEM_SHARED((NP, DH), jnp.float32),
            pltpu.VMEM_SHARED((NP, DEGW), jnp.float32),
            pltpu.SemaphoreType.DMA,
            pltpu.SemaphoreType.DMA,
        ],
        compiler_params=pltpu.CompilerParams(use_tc_tiling_on_sc=False),
    )
    def k(src_hbm, dst_hbm, feat_hbm, sum_out, deg_out,
          src_v, dst_v, rows_a, rows_b, ones_v, zsum_v, zdeg_v,
          acc_sh, deg_sh, sem_a, sem_b):
        cid = lax.axis_index("c")
        sid = lax.axis_index("s")
        row0 = sid * RPT

        zero16 = jnp.zeros((16,), jnp.float32)
        one16 = jnp.ones((16,), jnp.float32)

        def fill_ones(i, carry):
            ones_v[i] = one16
            return carry
        lax.fori_loop(0, CHUNK, fill_ones, 0)

        def fill_zsum(i, carry):
            r = i // (DH // 16)
            c = lax.rem(i, DH // 16)
            zsum_v[r, pl.ds(c * 16, 16)] = zero16
            return carry
        lax.fori_loop(0, ZROWS * (DH // 16), fill_zsum, 0)

        def fill_zdeg(i, carry):
            zdeg_v[i] = zero16
            return carry
        lax.fori_loop(0, RPT, fill_zdeg, 0)

        def zacc(t, carry):
            pltpu.sync_copy(zsum_v, acc_sh.at[pl.ds(row0 + t * ZROWS, ZROWS)])
            return carry
        lax.fori_loop(0, RPT // ZROWS, zacc, 0)
        pltpu.sync_copy(zdeg_v, deg_sh.at[pl.ds(row0, RPT)])

        pltpu.sync_copy(src_hbm.at[sid], src_v)
        pltpu.sync_copy(dst_hbm.at[sid], dst_v)

        plsc.subcore_barrier()

        def edge_loop(feat_ref, do_deg):
            def scat(buf, j):
                pltpu.sync_copy(buf, acc_sh.at[dst_v.at[j]], add=True)
                if do_deg:
                    pltpu.sync_copy(ones_v, deg_sh.at[dst_v.at[j]], add=True)

            pltpu.async_copy(feat_ref.at[src_v.at[0]], rows_a, sem_a)

            def body(jj, carry):
                j0 = 2 * jj
                j1 = j0 + 1
                pltpu.make_async_copy(feat_ref.at[src_v.at[j0]],
                                      rows_a, sem_a).wait()
                pltpu.async_copy(feat_ref.at[src_v.at[j1]], rows_b, sem_b)
                scat(rows_a, j0)
                pltpu.make_async_copy(feat_ref.at[src_v.at[j1]],
                                      rows_b, sem_b).wait()
                j2 = jnp.minimum(j1 + 1, NCHUNK - 1)
                pltpu.async_copy(feat_ref.at[src_v.at[j2]], rows_a, sem_a)
                scat(rows_b, j1)
                return carry
            lax.fori_loop(0, NCHUNK // 2, body, 0)
            pltpu.make_async_copy(feat_ref.at[src_v.at[NCHUNK - 1]],
                                  rows_a, sem_a).wait()

        @pl.when(cid == 0)
        def _():
            edge_loop(feat_hbm.at[0], True)

        @pl.when(cid == 1)
        def _():
            edge_loop(feat_hbm.at[1], False)

        plsc.subcore_barrier()

        pltpu.sync_copy(acc_sh.at[pl.ds(row0, RPT)],
                        sum_out.at[cid, pl.ds(row0, RPT)])

        @pl.when(cid == 0)
        def _():
            pltpu.sync_copy(deg_sh.at[pl.ds(row0, RPT)],
                            deg_out.at[pl.ds(row0, RPT)])

    return k(src, dst, feat_cols)


def _tc_finalize(sums, degs, W):
    R = 1000

    def body(s_ref, d_ref, w_ref, o_ref):
        s = jnp.concatenate([s_ref[0], s_ref[1]], axis=1)
        deg = d_ref[:, :1]
        rst = s / jnp.maximum(deg, 1.0)
        out = jnp.dot(rst, w_ref[...], preferred_element_type=jnp.float32)
        o_ref[...] = jnp.maximum(out, 0.0)

    return pl.pallas_call(
        body,
        grid=(N // R,),
        in_specs=[
            pl.BlockSpec((NC, R, DH), lambda i: (0, i, 0)),
            pl.BlockSpec((R, DEGW), lambda i: (i, 0)),
            pl.BlockSpec((D, D), lambda i: (0, 0)),
        ],
        out_specs=pl.BlockSpec((R, D), lambda i: (i, 0)),
        out_shape=jax.ShapeDtypeStruct((N, D), jnp.float32),
    )(sums, degs, W)


@jax.jit
def kernel(feat, edge_index, W):
    src = edge_index[0].reshape(NS, NCHUNK, CHUNK)
    dst = edge_index[1].reshape(NS, NCHUNK, CHUNK)
    feat_cols = jnp.stack([feat[:, :DH], feat[:, DH:]])
    sums, degs = _sc_aggregate(src, dst, feat_cols)
    return _tc_finalize(sums, degs, W)
